# baseline (speedup 1.0000x reference)
import jax
import jax.numpy as jnp
from jax.experimental import pallas as pl
from jax.experimental.pallas import tpu as pltpu

TOKEN_DIM = 768
INNER = 512
PLACEHOLDER_TOKEN = 42
N_TOKENS = 8192
BLOCK = 2048


def _body(tok_ref, emb_ref, lv_any, wv2_any, wo2_any, bo2_any, wnet_any,
          bnet_any, out_ref, ph_ref, lv_v, wv2_v, wo2_v, bo2_v, wnet_v,
          bnet_v, sems):
    i = pl.program_id(0)

    @pl.when(i == 0)
    def _compute_placeholder():
        copies = [
            pltpu.make_async_copy(lv_any, lv_v, sems.at[0]),
            pltpu.make_async_copy(wv2_any, wv2_v, sems.at[1]),
            pltpu.make_async_copy(wo2_any, wo2_v, sems.at[2]),
            pltpu.make_async_copy(bo2_any, bo2_v, sems.at[3]),
            pltpu.make_async_copy(wnet_any, wnet_v, sems.at[4]),
            pltpu.make_async_copy(bnet_any, bnet_v, sems.at[5]),
        ]
        for cp in copies:
            cp.start()
        for cp in copies:
            cp.wait()
        x = lv_v[...]
        v = jnp.dot(x, wv2_v[...], preferred_element_type=jnp.float32)
        x2 = jnp.dot(v, wo2_v[...], preferred_element_type=jnp.float32)
        x2 = x2 + bo2_v[...]
        ph = jnp.dot(x2, wnet_v[...], preferred_element_type=jnp.float32)
        ph_ref[...] = ph + bnet_v[...]

    mask = tok_ref[...] == PLACEHOLDER_TOKEN
    out_ref[...] = jnp.where(mask, ph_ref[...], emb_ref[...])


def kernel(tokenized_text, embedded_text, image_embeds, learnable_vector,
           Wq1, Wk1, Wv1, Wo1, bo1, Wq2, Wk2, Wv2, Wo2, bo2, Wnet, bnet):
    tok = tokenized_text.reshape(N_TOKENS, 1)
    emb = embedded_text.reshape(N_TOKENS, TOKEN_DIM)
    lv = learnable_vector.reshape(1, TOKEN_DIM)
    out = pl.pallas_call(
        _body,
        grid=(N_TOKENS // BLOCK,),
        in_specs=[
            pl.BlockSpec((BLOCK, 1), lambda i: (i, 0)),
            pl.BlockSpec((BLOCK, TOKEN_DIM), lambda i: (i, 0)),
            pl.BlockSpec(memory_space=pl.ANY),
            pl.BlockSpec(memory_space=pl.ANY),
            pl.BlockSpec(memory_space=pl.ANY),
            pl.BlockSpec(memory_space=pl.ANY),
            pl.BlockSpec(memory_space=pl.ANY),
            pl.BlockSpec(memory_space=pl.ANY),
        ],
        out_specs=pl.BlockSpec((BLOCK, TOKEN_DIM), lambda i: (i, 0)),
        out_shape=jax.ShapeDtypeStruct((N_TOKENS, TOKEN_DIM), jnp.float32),
        scratch_shapes=[
            pltpu.VMEM((1, TOKEN_DIM), jnp.float32),
            pltpu.VMEM((1, TOKEN_DIM), jnp.float32),
            pltpu.VMEM((TOKEN_DIM, INNER), jnp.float32),
            pltpu.VMEM((INNER, TOKEN_DIM), jnp.float32),
            pltpu.VMEM((1, TOKEN_DIM), jnp.float32),
            pltpu.VMEM((TOKEN_DIM, TOKEN_DIM), jnp.float32),
            pltpu.VMEM((1, TOKEN_DIM), jnp.float32),
            pltpu.SemaphoreType.DMA((6,)),
        ],
        compiler_params=pltpu.CompilerParams(
            dimension_semantics=("arbitrary",)),
    )(tok, emb, lv, Wv2, Wo2, bo2.reshape(1, TOKEN_DIM), Wnet,
      bnet.reshape(1, TOKEN_DIM))
    return out.reshape(1, N_TOKENS, TOKEN_DIM)


# speedup vs baseline: 1.0371x; 1.0371x over previous
import jax
import jax.numpy as jnp
from jax.experimental import pallas as pl
from jax.experimental.pallas import tpu as pltpu

TOKEN_DIM = 768
INNER = 512
PLACEHOLDER_TOKEN = 42
N_TOKENS = 8192
BLOCK = 2048


def _ph_body(lv_ref, wv2_ref, wo2_ref, bo2_ref, wnet_ref, bnet_ref, ph_ref):
    x = lv_ref[...]
    v = jnp.dot(x, wv2_ref[...], preferred_element_type=jnp.float32)
    x2 = jnp.dot(v, wo2_ref[...], preferred_element_type=jnp.float32)
    x2 = x2 + bo2_ref[...]
    ph = jnp.dot(x2, wnet_ref[...], preferred_element_type=jnp.float32)
    ph_ref[...] = ph + bnet_ref[...]


def _select_body(tok_ref, emb_ref, ph_ref, out_ref):
    mask = tok_ref[...] == PLACEHOLDER_TOKEN
    out_ref[...] = jnp.where(mask, ph_ref[...], emb_ref[...])


def kernel(tokenized_text, embedded_text, image_embeds, learnable_vector,
           Wq1, Wk1, Wv1, Wo1, bo1, Wq2, Wk2, Wv2, Wo2, bo2, Wnet, bnet):
    tok = tokenized_text.reshape(N_TOKENS, 1)
    emb = embedded_text.reshape(N_TOKENS, TOKEN_DIM)
    lv = learnable_vector.reshape(1, TOKEN_DIM)
    ph = pl.pallas_call(
        _ph_body,
        out_shape=jax.ShapeDtypeStruct((1, TOKEN_DIM), jnp.float32),
    )(lv, Wv2, Wo2, bo2.reshape(1, TOKEN_DIM), Wnet,
      bnet.reshape(1, TOKEN_DIM))
    out = pl.pallas_call(
        _select_body,
        grid=(N_TOKENS // BLOCK,),
        in_specs=[
            pl.BlockSpec((BLOCK, 1), lambda i: (i, 0)),
            pl.BlockSpec((BLOCK, TOKEN_DIM), lambda i: (i, 0)),
            pl.BlockSpec((1, TOKEN_DIM), lambda i: (0, 0)),
        ],
        out_specs=pl.BlockSpec((BLOCK, TOKEN_DIM), lambda i: (i, 0)),
        out_shape=jax.ShapeDtypeStruct((N_TOKENS, TOKEN_DIM), jnp.float32),
        compiler_params=pltpu.CompilerParams(
            dimension_semantics=("parallel",)),
    )(tok, emb, ph)
    return out.reshape(1, N_TOKENS, TOKEN_DIM)
